# baseline (speedup 1.0000x reference)
import jax
import jax.numpy as jnp
import numpy as np
from jax.experimental import pallas as pl
from jax.experimental.pallas import tpu as pltpu

B, N, NBH = 8, 512, 32
NIN, NF, NOUT, NG, HEADS = 128, 128, 128, 64, 4
BN = 256
BNK = BN * NBH
LOG2 = float(np.log(2.0))


def _ssp(v):
    return jax.nn.softplus(v) - LOG2


def _body(x_ref, r_ref, nbh_ref, mask_ref, win_ref, wst_ref,
          fw1_ref, fb1_ref, fw2_ref, fb2_ref, ow_ref, ob_ref, out_ref,
          tab_ref, st_ref):
    nb = pl.program_id(1)

    @pl.when(nb == 0)
    def _():
        xb = x_ref[0]
        ys = jnp.dot(xb, win_ref[...],
                     preferred_element_type=jnp.float32)
        st = jnp.dot(ys, wst_ref[...],
                     preferred_element_type=jnp.float32)
        st_ref[...] = st
        tab_ref[...] = jnp.concatenate(
            [ys.astype(jnp.bfloat16), st[:, HEADS:].astype(jnp.bfloat16)],
            axis=1)

    s_blk = st_ref[pl.ds(nb * BN, BN), :HEADS]

    nbhb = nbh_ref[0]
    iota = jax.lax.broadcasted_iota(jnp.int32, (BN, NBH, N), 2)
    oh = (nbhb[:, :, None] == iota).astype(jnp.bfloat16).reshape(BNK, N)

    yg = jnp.dot(oh, tab_ref[...],
                 preferred_element_type=jnp.float32)
    tg = yg[:, HEADS * NF:].reshape(BN, NBH, HEADS)

    att = s_blk[:, None, :] + tg
    e = 1.0 + jnp.exp(att)
    alpha = e / jnp.sum(e, axis=1, keepdims=True)
    coef = alpha * mask_ref[0][:, :, None] * (1.0 / HEADS)
    cf = coef.reshape(BNK, HEADS)
    g = jnp.zeros((BNK, NF), dtype=jnp.float32)
    for i in range(HEADS):
        g = g + yg[:, i * NF:(i + 1) * NF] * cf[:, i:i + 1]

    rr = r_ref[0]
    h = _ssp(jnp.dot(rr, fw1_ref[...], preferred_element_type=jnp.float32)
             + fb1_ref[...])
    wf = jnp.dot(h, fw2_ref[...], preferred_element_type=jnp.float32) \
        + fb2_ref[...]

    pre = (wf * g).reshape(BN, NBH, NF).sum(axis=1)
    out_ref[0] = _ssp(jnp.dot(pre, ow_ref[...],
                              preferred_element_type=jnp.float32) + ob_ref[...])


@jax.jit
def kernel(x, r_ij, neighbors, pairwise_mask, W_in, W_att, fW1, fb1, fW2, fb2,
           oW, ob):
    win_s = jnp.transpose(W_in, (1, 0, 2)).reshape(NIN, HEADS * NF)
    wa = W_att[:, :, 0]
    wst = jnp.zeros((HEADS * NF, 2 * HEADS), dtype=jnp.float32)
    for i in range(HEADS):
        wst = wst.at[i * NF:(i + 1) * NF, i].set(wa[i, :NF])
        wst = wst.at[i * NF:(i + 1) * NF, HEADS + i].set(wa[i, NF:])

    r2 = r_ij.reshape(B, N * NBH, NG).astype(jnp.bfloat16)
    nbh = neighbors.astype(jnp.int32)
    grid = (B, N // BN)

    out = pl.pallas_call(
        _body,
        grid=grid,
        in_specs=[
            pl.BlockSpec((1, N, NIN), lambda b, nb: (b, 0, 0)),
            pl.BlockSpec((1, BNK, NG), lambda b, nb: (b, nb, 0)),
            pl.BlockSpec((1, BN, NBH), lambda b, nb: (b, nb, 0)),
            pl.BlockSpec((1, BN, NBH), lambda b, nb: (b, nb, 0)),
            pl.BlockSpec((NIN, HEADS * NF), lambda b, nb: (0, 0)),
            pl.BlockSpec((HEADS * NF, 2 * HEADS), lambda b, nb: (0, 0)),
            pl.BlockSpec((NG, NF), lambda b, nb: (0, 0)),
            pl.BlockSpec((1, NF), lambda b, nb: (0, 0)),
            pl.BlockSpec((NF, NF), lambda b, nb: (0, 0)),
            pl.BlockSpec((1, NF), lambda b, nb: (0, 0)),
            pl.BlockSpec((NF, NOUT), lambda b, nb: (0, 0)),
            pl.BlockSpec((1, NOUT), lambda b, nb: (0, 0)),
        ],
        out_specs=pl.BlockSpec((1, BN, NOUT), lambda b, nb: (b, nb, 0)),
        out_shape=jax.ShapeDtypeStruct((B, N, NOUT), jnp.float32),
        scratch_shapes=[
            pltpu.VMEM((N, HEADS * NF + HEADS), jnp.bfloat16),
            pltpu.VMEM((N, 2 * HEADS), jnp.float32),
        ],
        compiler_params=pltpu.CompilerParams(
            dimension_semantics=("arbitrary", "arbitrary"),
        ),
    )(x, r2, nbh, pairwise_mask, win_s, wst, fW1.astype(jnp.bfloat16), fb1.reshape(1, NF), fW2,
      fb2.reshape(1, NF), oW, ob.reshape(1, NOUT))
    return out


# speedup vs baseline: 1.0172x; 1.0172x over previous
import jax
import jax.numpy as jnp
import numpy as np
from jax.experimental import pallas as pl
from jax.experimental.pallas import tpu as pltpu

B, N, NBH = 8, 512, 32
NIN, NF, NOUT, NG, HEADS = 128, 128, 128, 64, 4
BN = 256
BNK = BN * NBH
LOG2 = float(np.log(2.0))


def _ssp(v):
    return jax.nn.softplus(v) - LOG2


def _body(x_ref, r_ref, nbh_ref, mask_ref, win_ref, wa_ref,
          fw1_ref, fb1_ref, fw2_ref, fb2_ref, ow_ref, ob_ref, out_ref,
          tab_ref, s_ref):
    nb = pl.program_id(1)

    @pl.when(nb == 0)
    def _():
        xb = x_ref[0]
        for i in range(HEADS):
            yi = jnp.dot(xb, win_ref[i],
                         preferred_element_type=jnp.float32)
            tab_ref[:, i * NF:(i + 1) * NF] = yi.astype(jnp.bfloat16)
            si = jnp.dot(yi, wa_ref[i, :NF],
                         preferred_element_type=jnp.float32)
            ti = jnp.dot(yi, wa_ref[i, NF:],
                         preferred_element_type=jnp.float32)
            s_ref[:, i:i + 1] = si
            tab_ref[:, HEADS * NF + i:HEADS * NF + i + 1] = \
                ti.astype(jnp.bfloat16)

    s_blk = s_ref[pl.ds(nb * BN, BN), :]

    nbhb = nbh_ref[0]
    iota = jax.lax.broadcasted_iota(jnp.int32, (BN, NBH, N), 2)
    oh = (nbhb[:, :, None] == iota).astype(jnp.bfloat16).reshape(BNK, N)

    yg = jnp.dot(oh, tab_ref[...],
                 preferred_element_type=jnp.float32)
    tg = yg[:, HEADS * NF:].reshape(BN, NBH, HEADS)

    att = s_blk[:, None, :] + tg
    e = 1.0 + jnp.exp(att)
    alpha = e / jnp.sum(e, axis=1, keepdims=True)
    coef = alpha * mask_ref[0][:, :, None] * (1.0 / HEADS)
    cf = coef.reshape(BNK, HEADS)
    g = jnp.zeros((BNK, NF), dtype=jnp.float32)
    for i in range(HEADS):
        g = g + yg[:, i * NF:(i + 1) * NF] * cf[:, i:i + 1]

    rr = r_ref[0]
    h = _ssp(jnp.dot(rr, fw1_ref[...], preferred_element_type=jnp.float32)
             + fb1_ref[...])
    wf = jnp.dot(h, fw2_ref[...], preferred_element_type=jnp.float32) \
        + fb2_ref[...]

    pre = (wf * g).reshape(BN, NBH, NF).sum(axis=1)
    out_ref[0] = _ssp(jnp.dot(pre, ow_ref[...],
                              preferred_element_type=jnp.float32) + ob_ref[...])


@jax.jit
def kernel(x, r_ij, neighbors, pairwise_mask, W_in, W_att, fW1, fb1, fW2, fb2,
           oW, ob):
    r2 = r_ij.reshape(B, N * NBH, NG)
    nbh = neighbors.astype(jnp.int32)
    wa = W_att.reshape(HEADS, 2 * NF, 1)
    grid = (B, N // BN)

    out = pl.pallas_call(
        _body,
        grid=grid,
        in_specs=[
            pl.BlockSpec((1, N, NIN), lambda b, nb: (b, 0, 0)),
            pl.BlockSpec((1, BNK, NG), lambda b, nb: (b, nb, 0)),
            pl.BlockSpec((1, BN, NBH), lambda b, nb: (b, nb, 0)),
            pl.BlockSpec((1, BN, NBH), lambda b, nb: (b, nb, 0)),
            pl.BlockSpec((HEADS, NIN, NF), lambda b, nb: (0, 0, 0)),
            pl.BlockSpec((HEADS, 2 * NF, 1), lambda b, nb: (0, 0, 0)),
            pl.BlockSpec((NG, NF), lambda b, nb: (0, 0)),
            pl.BlockSpec((1, NF), lambda b, nb: (0, 0)),
            pl.BlockSpec((NF, NF), lambda b, nb: (0, 0)),
            pl.BlockSpec((1, NF), lambda b, nb: (0, 0)),
            pl.BlockSpec((NF, NOUT), lambda b, nb: (0, 0)),
            pl.BlockSpec((1, NOUT), lambda b, nb: (0, 0)),
        ],
        out_specs=pl.BlockSpec((1, BN, NOUT), lambda b, nb: (b, nb, 0)),
        out_shape=jax.ShapeDtypeStruct((B, N, NOUT), jnp.float32),
        scratch_shapes=[
            pltpu.VMEM((N, HEADS * NF + HEADS), jnp.bfloat16),
            pltpu.VMEM((N, HEADS), jnp.float32),
        ],
        compiler_params=pltpu.CompilerParams(
            dimension_semantics=("arbitrary", "arbitrary"),
        ),
    )(x, r2, nbh, pairwise_mask, W_in, wa, fW1, fb1.reshape(1, NF), fW2,
      fb2.reshape(1, NF), oW, ob.reshape(1, NOUT))
    return out


# speedup vs baseline: 1.0481x; 1.0304x over previous
import jax
import jax.numpy as jnp
import numpy as np
from jax.experimental import pallas as pl
from jax.experimental.pallas import tpu as pltpu

B, N, NBH = 8, 512, 32
NIN, NF, NOUT, NG, HEADS = 128, 128, 128, 64, 4
BN = 256
BNK = BN * NBH
LOG2 = float(np.log(2.0))


def _ssp(v):
    return jax.nn.softplus(v) - LOG2


def _body(x_ref, xblk_ref, r_ref, nbh_ref, mask_ref, win_ref, wst_ref,
          fw1_ref, fb1_ref, fw2_ref, fb2_ref, ow_ref, ob_ref, out_ref):
    xb = x_ref[0]
    ys = jnp.dot(xb, win_ref[...], preferred_element_type=jnp.float32)
    st = jnp.dot(ys, wst_ref[...], preferred_element_type=jnp.float32)
    ys_blk = jnp.dot(xblk_ref[0], win_ref[...],
                     preferred_element_type=jnp.float32)
    s_blk = jnp.dot(ys_blk, wst_ref[...],
                    preferred_element_type=jnp.float32)[:, :HEADS]

    nbhb = nbh_ref[0]
    iota = jax.lax.broadcasted_iota(jnp.int32, (BN, NBH, N), 2)
    oh = (nbhb[:, :, None] == iota).astype(jnp.bfloat16).reshape(BNK, N)

    tab = jnp.concatenate(
        [ys.astype(jnp.bfloat16), st[:, HEADS:].astype(jnp.bfloat16)], axis=1)
    yg = jnp.dot(oh, tab, preferred_element_type=jnp.float32)
    tg = yg[:, HEADS * NF:].reshape(BN, NBH, HEADS)

    att = s_blk[:, None, :] + tg
    e = 1.0 + jnp.exp(att)
    alpha = e / jnp.sum(e, axis=1, keepdims=True)
    coef = alpha * mask_ref[0][:, :, None] * (1.0 / HEADS)
    cf = coef.reshape(BNK, HEADS)
    g = jnp.zeros((BNK, NF), dtype=jnp.float32)
    for i in range(HEADS):
        g = g + yg[:, i * NF:(i + 1) * NF] * cf[:, i:i + 1]

    rr = r_ref[0]
    h = _ssp(jnp.dot(rr, fw1_ref[...], preferred_element_type=jnp.float32)
             + fb1_ref[...])
    wf = jnp.dot(h, fw2_ref[...], preferred_element_type=jnp.float32) \
        + fb2_ref[...]

    pre = (wf * g).reshape(BN, NBH, NF).sum(axis=1)
    out_ref[0] = _ssp(jnp.dot(pre, ow_ref[...],
                              preferred_element_type=jnp.float32) + ob_ref[...])


@jax.jit
def kernel(x, r_ij, neighbors, pairwise_mask, W_in, W_att, fW1, fb1, fW2, fb2,
           oW, ob):
    win_s = jnp.transpose(W_in, (1, 0, 2)).reshape(NIN, HEADS * NF)
    wa = W_att[:, :, 0]
    wst = jnp.zeros((HEADS * NF, 2 * HEADS), dtype=jnp.float32)
    for i in range(HEADS):
        wst = wst.at[i * NF:(i + 1) * NF, i].set(wa[i, :NF])
        wst = wst.at[i * NF:(i + 1) * NF, HEADS + i].set(wa[i, NF:])

    r2 = r_ij.reshape(B, N * NBH, NG)
    nbh = neighbors.astype(jnp.int32)
    grid = (B, N // BN)

    out = pl.pallas_call(
        _body,
        grid=grid,
        in_specs=[
            pl.BlockSpec((1, N, NIN), lambda b, nb: (b, 0, 0)),
            pl.BlockSpec((1, BN, NIN), lambda b, nb: (b, nb, 0)),
            pl.BlockSpec((1, BNK, NG), lambda b, nb: (b, nb, 0)),
            pl.BlockSpec((1, BN, NBH), lambda b, nb: (b, nb, 0)),
            pl.BlockSpec((1, BN, NBH), lambda b, nb: (b, nb, 0)),
            pl.BlockSpec((NIN, HEADS * NF), lambda b, nb: (0, 0)),
            pl.BlockSpec((HEADS * NF, 2 * HEADS), lambda b, nb: (0, 0)),
            pl.BlockSpec((NG, NF), lambda b, nb: (0, 0)),
            pl.BlockSpec((1, NF), lambda b, nb: (0, 0)),
            pl.BlockSpec((NF, NF), lambda b, nb: (0, 0)),
            pl.BlockSpec((1, NF), lambda b, nb: (0, 0)),
            pl.BlockSpec((NF, NOUT), lambda b, nb: (0, 0)),
            pl.BlockSpec((1, NOUT), lambda b, nb: (0, 0)),
        ],
        out_specs=pl.BlockSpec((1, BN, NOUT), lambda b, nb: (b, nb, 0)),
        out_shape=jax.ShapeDtypeStruct((B, N, NOUT), jnp.float32),
        compiler_params=pltpu.CompilerParams(
            dimension_semantics=("parallel", "arbitrary"),
        ),
    )(x, x, r2, nbh, pairwise_mask, win_s, wst, fW1, fb1.reshape(1, NF), fW2,
      fb2.reshape(1, NF), oW, ob.reshape(1, NOUT))
    return out
